# baseline (speedup 1.0000x reference)
import functools

import jax
import jax.numpy as jnp
from jax import lax
from jax.experimental import pallas as pl
from jax.experimental.pallas import tpu as pltpu
from jax.experimental.pallas import tpu_sc as plsc

N_FEATURES = 32768
D_MODEL = 2048
BATCH = 16384
K = 32
GRAD_CLIP_THRESHOLD = 1.0

NW = 32
ROWS_W = BATCH // NW
_UNROLL = 8


def _make_hist_kernel():
    mesh = plsc.VectorSubcoreMesh(core_axis_name="c", subcore_axis_name="s")

    @functools.partial(
        pl.kernel,
        out_type=jax.ShapeDtypeStruct((NW * N_FEATURES,), jnp.float32),
        mesh=mesh,
        scratch_types=[
            pltpu.VMEM((ROWS_W, K), jnp.int32),
            pltpu.VMEM((N_FEATURES,), jnp.float32),
        ],
        compiler_params=pltpu.CompilerParams(
            needs_layout_passes=False, skip_device_barrier=True
        ),
    )
    def hist_kernel(idx_hbm, out_hbm, idx_v, hist_v):
        wid = lax.axis_index("s") * 2 + lax.axis_index("c")
        zero = jnp.zeros((16,), jnp.float32)

        def zbody(i, c):
            for j in range(_UNROLL):
                hist_v[pl.ds((i * _UNROLL + j) * 16, 16)] = zero
            return c

        lax.fori_loop(0, N_FEATURES // (16 * _UNROLL), zbody, 0)

        pltpu.sync_copy(idx_hbm.at[pl.ds(wid * ROWS_W, ROWS_W), :], idx_v)

        ones = jnp.ones((16,), jnp.float32)

        def body(i, c):
            for r in range(4):
                for j in range(K // 16):
                    vec = idx_v[i * 4 + r, pl.ds(j * 16, 16)]
                    plsc.addupdate_scatter(hist_v, [vec], ones)
            return c

        lax.fori_loop(0, ROWS_W // 4, body, 0)

        pltpu.sync_copy(hist_v, out_hbm.at[pl.ds(wid * N_FEATURES, N_FEATURES)])

    return hist_kernel


_X_BLOCK = 1024
_U_BLOCK = 512
_GRID = BATCH // _X_BLOCK


def _reduce_body(
    w_ref, nw_ref, avg_ref, gcp_ref, x_ref, u_ref, norm_ref, clip_ref, acc_ref
):
    i = pl.program_id(0)

    @pl.when(i == 0)
    def _init():
        acc_ref[0] = 0.0
        acc_ref[1] = 0.0

    xb = x_ref[...]
    rs = jnp.sum(xb * xb, axis=1, keepdims=True)
    nsum = jnp.sum(jnp.sqrt(rs))
    ub = u_ref[...]
    csum = jnp.sum((jnp.abs(ub) > GRAD_CLIP_THRESHOLD).astype(jnp.float32))
    acc_ref[0] += nsum
    acc_ref[1] += csum

    @pl.when(i == _GRID - 1)
    def _fini():
        w = w_ref[0]
        nw = nw_ref[0]
        norm_ref[0] = avg_ref[0] * w + (acc_ref[0] / BATCH) * nw
        clip_ref[0] = gcp_ref[0] * w + (
            acc_ref[1] / (8192.0 * D_MODEL)
        ) * nw


def _dense_reduce(w, nw, avg_norm, gcp, x, updates_flat):
    return pl.pallas_call(
        _reduce_body,
        grid=(_GRID,),
        in_specs=[
            pl.BlockSpec(memory_space=pltpu.SMEM),
            pl.BlockSpec(memory_space=pltpu.SMEM),
            pl.BlockSpec(memory_space=pltpu.SMEM),
            pl.BlockSpec(memory_space=pltpu.SMEM),
            pl.BlockSpec((_X_BLOCK, D_MODEL), lambda i: (i, 0)),
            pl.BlockSpec((_U_BLOCK, D_MODEL), lambda i: (i, 0)),
        ],
        out_specs=[
            pl.BlockSpec(memory_space=pltpu.SMEM),
            pl.BlockSpec(memory_space=pltpu.SMEM),
        ],
        out_shape=[
            jax.ShapeDtypeStruct((1,), jnp.float32),
            jax.ShapeDtypeStruct((1,), jnp.float32),
        ],
        scratch_shapes=[pltpu.SMEM((2,), jnp.float32)],
        compiler_params=pltpu.CompilerParams(
            dimension_semantics=("arbitrary",)
        ),
    )(w, nw, avg_norm, gcp, x, updates_flat)


def _blend_body(w_ref, nw_ref, fd_ref, h_ref, out_ref, acc_ref):
    i = pl.program_id(0)

    @pl.when(i == 0)
    def _init():
        acc_ref[...] = jnp.zeros((N_FEATURES,), jnp.float32)

    acc_ref[...] += h_ref[...]

    @pl.when(i == NW - 1)
    def _fini():
        out_ref[...] = fd_ref[...] * w_ref[0] + acc_ref[...] * nw_ref[0]


def _blend(w, nw, fd, hists1d):
    return pl.pallas_call(
        _blend_body,
        grid=(NW,),
        in_specs=[
            pl.BlockSpec(memory_space=pltpu.SMEM),
            pl.BlockSpec(memory_space=pltpu.SMEM),
            pl.BlockSpec((N_FEATURES,), lambda i: (0,)),
            pl.BlockSpec((N_FEATURES,), lambda i: (i,)),
        ],
        out_specs=pl.BlockSpec((N_FEATURES,), lambda i: (0,)),
        out_shape=jax.ShapeDtypeStruct((N_FEATURES,), jnp.float32),
        scratch_shapes=[pltpu.VMEM((N_FEATURES,), jnp.float32)],
        compiler_params=pltpu.CompilerParams(
            dimension_semantics=("arbitrary",)
        ),
    )(w, nw, fd, hists1d)


def kernel(n_steps, avg_norm, feature_density, grad_clip_percent, updates_flat, x, k_indices):
    ns = jnp.asarray(n_steps, jnp.float32)
    w = (ns / (ns + 1.0)).reshape(1)
    nw = (1.0 / (ns + 1.0)).reshape(1)

    hist_kernel = _make_hist_kernel()
    hists1d = hist_kernel(k_indices)

    updated_avg_norm, updated_clip = _dense_reduce(
        w,
        nw,
        jnp.asarray(avg_norm, jnp.float32).reshape(1),
        jnp.asarray(grad_clip_percent, jnp.float32).reshape(1),
        x,
        updates_flat,
    )

    updated_fd = _blend(w, nw, feature_density, hists1d)

    return (
        jnp.asarray(n_steps + 1),
        updated_avg_norm[0],
        updated_fd,
        updated_clip[0],
    )


# speedup vs baseline: 1.0016x; 1.0016x over previous
import functools

import jax
import jax.numpy as jnp
from jax import lax
from jax.experimental import pallas as pl
from jax.experimental.pallas import tpu as pltpu
from jax.experimental.pallas import tpu_sc as plsc

N_FEATURES = 32768
D_MODEL = 2048
BATCH = 16384
K = 32
GRAD_CLIP_THRESHOLD = 1.0

NW = 32
ROWS_W = BATCH // NW
_UNROLL = 8


def _make_hist_kernel():
    mesh = plsc.VectorSubcoreMesh(core_axis_name="c", subcore_axis_name="s")

    @functools.partial(
        pl.kernel,
        out_type=jax.ShapeDtypeStruct((NW * N_FEATURES,), jnp.float32),
        mesh=mesh,
        scratch_types=[
            pltpu.VMEM((ROWS_W, K), jnp.int32),
            pltpu.VMEM((N_FEATURES,), jnp.float32),
        ],
        compiler_params=pltpu.CompilerParams(
            needs_layout_passes=False, skip_device_barrier=True
        ),
    )
    def hist_kernel(idx_hbm, out_hbm, idx_v, hist_v):
        wid = lax.axis_index("s") * 2 + lax.axis_index("c")
        zero = jnp.zeros((16,), jnp.float32)

        def zbody(i, c):
            for j in range(_UNROLL):
                hist_v[pl.ds((i * _UNROLL + j) * 16, 16)] = zero
            return c

        lax.fori_loop(0, N_FEATURES // (16 * _UNROLL), zbody, 0)

        pltpu.sync_copy(idx_hbm.at[pl.ds(wid * ROWS_W, ROWS_W), :], idx_v)

        ones = jnp.ones((16,), jnp.float32)

        def body(i, c):
            for r in range(4):
                for j in range(K // 16):
                    vec = idx_v[i * 4 + r, pl.ds(j * 16, 16)]
                    plsc.addupdate_scatter(hist_v, [vec], ones)
            return c

        lax.fori_loop(0, ROWS_W // 4, body, 0)

        pltpu.sync_copy(hist_v, out_hbm.at[pl.ds(wid * N_FEATURES, N_FEATURES)])

    return hist_kernel


_X_BLOCK = 1024
_U_BLOCK = 512
_GRID = BATCH // _X_BLOCK


def _reduce_body(
    w_ref, nw_ref, avg_ref, gcp_ref, x_ref, u_ref, norm_ref, clip_ref, acc_ref
):
    i = pl.program_id(0)

    @pl.when(i == 0)
    def _init():
        acc_ref[0] = 0.0
        acc_ref[1] = 0.0

    xb = x_ref[...]
    rs = jnp.sum(xb * xb, axis=1, keepdims=True)
    nsum = jnp.sum(jnp.sqrt(rs))
    ub = u_ref[...]
    csum = jnp.sum((jnp.abs(ub) > GRAD_CLIP_THRESHOLD).astype(jnp.float32))
    acc_ref[0] += nsum
    acc_ref[1] += csum

    @pl.when(i == _GRID - 1)
    def _fini():
        w = w_ref[0]
        nw = nw_ref[0]
        norm_ref[0] = avg_ref[0] * w + (acc_ref[0] / BATCH) * nw
        clip_ref[0] = gcp_ref[0] * w + (
            acc_ref[1] / (8192.0 * D_MODEL)
        ) * nw


def _dense_reduce(w, nw, avg_norm, gcp, x, updates_flat):
    return pl.pallas_call(
        _reduce_body,
        grid=(_GRID,),
        in_specs=[
            pl.BlockSpec(memory_space=pltpu.SMEM),
            pl.BlockSpec(memory_space=pltpu.SMEM),
            pl.BlockSpec(memory_space=pltpu.SMEM),
            pl.BlockSpec(memory_space=pltpu.SMEM),
            pl.BlockSpec((_X_BLOCK, D_MODEL), lambda i: (i, 0)),
            pl.BlockSpec((_U_BLOCK, D_MODEL), lambda i: (i, 0)),
        ],
        out_specs=[
            pl.BlockSpec(memory_space=pltpu.SMEM),
            pl.BlockSpec(memory_space=pltpu.SMEM),
        ],
        out_shape=[
            jax.ShapeDtypeStruct((1,), jnp.float32),
            jax.ShapeDtypeStruct((1,), jnp.float32),
        ],
        scratch_shapes=[pltpu.SMEM((2,), jnp.float32)],
        compiler_params=pltpu.CompilerParams(
            dimension_semantics=("arbitrary",)
        ),
    )(w, nw, avg_norm, gcp, x, updates_flat)


_FD_ROWS = N_FEATURES // 128


def _blend_body(w_ref, nw_ref, fd_ref, h_ref, out_ref, acc_ref):
    i = pl.program_id(0)

    @pl.when(i == 0)
    def _init():
        acc_ref[...] = jnp.zeros((_FD_ROWS, 128), jnp.float32)

    acc_ref[...] += h_ref[...]

    @pl.when(i == NW - 1)
    def _fini():
        out_ref[...] = fd_ref[...] * w_ref[0] + acc_ref[...] * nw_ref[0]


def _blend(w, nw, fd2, hists2):
    return pl.pallas_call(
        _blend_body,
        grid=(NW,),
        in_specs=[
            pl.BlockSpec(memory_space=pltpu.SMEM),
            pl.BlockSpec(memory_space=pltpu.SMEM),
            pl.BlockSpec((_FD_ROWS, 128), lambda i: (0, 0)),
            pl.BlockSpec((_FD_ROWS, 128), lambda i: (i, 0)),
        ],
        out_specs=pl.BlockSpec((_FD_ROWS, 128), lambda i: (0, 0)),
        out_shape=jax.ShapeDtypeStruct((_FD_ROWS, 128), jnp.float32),
        scratch_shapes=[pltpu.VMEM((_FD_ROWS, 128), jnp.float32)],
        compiler_params=pltpu.CompilerParams(
            dimension_semantics=("arbitrary",)
        ),
    )(w, nw, fd2, hists2)


def kernel(n_steps, avg_norm, feature_density, grad_clip_percent, updates_flat, x, k_indices):
    ns = jnp.asarray(n_steps, jnp.float32)
    w = (ns / (ns + 1.0)).reshape(1)
    nw = (1.0 / (ns + 1.0)).reshape(1)

    hist_kernel = _make_hist_kernel()
    hists1d = hist_kernel(k_indices)

    updated_avg_norm, updated_clip = _dense_reduce(
        w,
        nw,
        jnp.asarray(avg_norm, jnp.float32).reshape(1),
        jnp.asarray(grad_clip_percent, jnp.float32).reshape(1),
        x,
        updates_flat,
    )

    fd2 = feature_density.reshape(_FD_ROWS, 128)
    hists2 = hists1d.reshape(NW * _FD_ROWS, 128)
    updated_fd = _blend(w, nw, fd2, hists2).reshape(N_FEATURES)

    return (
        jnp.asarray(n_steps + 1),
        updated_avg_norm[0],
        updated_fd,
        updated_clip[0],
    )


# speedup vs baseline: 1.1401x; 1.1383x over previous
import functools

import jax
import jax.numpy as jnp
from jax import lax
from jax.experimental import pallas as pl
from jax.experimental.pallas import tpu as pltpu
from jax.experimental.pallas import tpu_sc as plsc

N_FEATURES = 32768
D_MODEL = 2048
BATCH = 16384
K = 32
GRAD_CLIP_THRESHOLD = 1.0

NW = 32
ROWS_W = BATCH // NW
_UNROLL = 8


def _make_hist_kernel():
    mesh = plsc.VectorSubcoreMesh(core_axis_name="c", subcore_axis_name="s")

    @functools.partial(
        pl.kernel,
        out_type=jax.ShapeDtypeStruct((NW * N_FEATURES,), jnp.float32),
        mesh=mesh,
        scratch_types=[
            pltpu.VMEM((ROWS_W, K), jnp.int32),
            pltpu.VMEM((N_FEATURES,), jnp.float32),
        ],
        compiler_params=pltpu.CompilerParams(
            needs_layout_passes=False, skip_device_barrier=True
        ),
    )
    def hist_kernel(idx_hbm, out_hbm, idx_v, hist_v):
        wid = lax.axis_index("s") * 2 + lax.axis_index("c")
        zero = jnp.zeros((16,), jnp.float32)

        def zbody(i, c):
            for j in range(_UNROLL):
                hist_v[pl.ds((i * _UNROLL + j) * 16, 16)] = zero
            return c

        lax.fori_loop(0, N_FEATURES // (16 * _UNROLL), zbody, 0)

        pltpu.sync_copy(idx_hbm.at[pl.ds(wid * ROWS_W, ROWS_W), :], idx_v)

        ones = jnp.ones((16,), jnp.float32)

        def body(i, c):
            for r in range(4):
                for j in range(K // 16):
                    vec = idx_v[i * 4 + r, pl.ds(j * 16, 16)]
                    plsc.addupdate_scatter(hist_v, [vec], ones)
            return c

        lax.fori_loop(0, ROWS_W // 4, body, 0)

        pltpu.sync_copy(hist_v, out_hbm.at[pl.ds(wid * N_FEATURES, N_FEATURES)])

    return hist_kernel


_X_BLOCK = 1024
_U_BLOCK = 512
_GRID = BATCH // _X_BLOCK


def _reduce_body(
    w_ref, nw_ref, avg_ref, gcp_ref, x_ref, u_ref, norm_ref, clip_ref, acc_ref
):
    i = pl.program_id(0)

    @pl.when(i == 0)
    def _init():
        acc_ref[0] = 0.0
        acc_ref[1] = 0.0

    xb = x_ref[...]
    rs = jnp.sum(xb * xb, axis=1, keepdims=True)
    nsum = jnp.sum(jnp.sqrt(rs))
    ub = u_ref[...]
    csum = jnp.sum((jnp.abs(ub) > GRAD_CLIP_THRESHOLD).astype(jnp.float32))
    acc_ref[0] += nsum
    acc_ref[1] += csum

    @pl.when(i == _GRID - 1)
    def _fini():
        w = w_ref[0]
        nw = nw_ref[0]
        norm_ref[0] = avg_ref[0] * w + (acc_ref[0] / BATCH) * nw
        clip_ref[0] = gcp_ref[0] * w + (
            acc_ref[1] / (8192.0 * D_MODEL)
        ) * nw


def _dense_reduce(w, nw, avg_norm, gcp, x, updates_flat):
    return pl.pallas_call(
        _reduce_body,
        grid=(_GRID,),
        in_specs=[
            pl.BlockSpec(memory_space=pltpu.SMEM),
            pl.BlockSpec(memory_space=pltpu.SMEM),
            pl.BlockSpec(memory_space=pltpu.SMEM),
            pl.BlockSpec(memory_space=pltpu.SMEM),
            pl.BlockSpec((_X_BLOCK, D_MODEL), lambda i: (i, 0)),
            pl.BlockSpec((_U_BLOCK, D_MODEL), lambda i: (i, 0)),
        ],
        out_specs=[
            pl.BlockSpec(memory_space=pltpu.SMEM),
            pl.BlockSpec(memory_space=pltpu.SMEM),
        ],
        out_shape=[
            jax.ShapeDtypeStruct((1,), jnp.float32),
            jax.ShapeDtypeStruct((1,), jnp.float32),
        ],
        scratch_shapes=[pltpu.SMEM((2,), jnp.float32)],
        compiler_params=pltpu.CompilerParams(
            dimension_semantics=("arbitrary",)
        ),
    )(w, nw, avg_norm, gcp, x, updates_flat)


_FD_ROWS = N_FEATURES // 128


def _blend_body(w_ref, nw_ref, fd_ref, h_ref, out_ref):
    h = h_ref[...].reshape(NW, _FD_ROWS, 128)
    tot = jnp.sum(h, axis=0)
    out_ref[...] = fd_ref[...] * w_ref[0] + tot * nw_ref[0]


def _blend(w, nw, fd2, hists2):
    return pl.pallas_call(
        _blend_body,
        in_specs=[
            pl.BlockSpec(memory_space=pltpu.SMEM),
            pl.BlockSpec(memory_space=pltpu.SMEM),
            pl.BlockSpec(memory_space=pltpu.VMEM),
            pl.BlockSpec(memory_space=pltpu.VMEM),
        ],
        out_shape=jax.ShapeDtypeStruct((_FD_ROWS, 128), jnp.float32),
    )(w, nw, fd2, hists2)


def kernel(n_steps, avg_norm, feature_density, grad_clip_percent, updates_flat, x, k_indices):
    ns = jnp.asarray(n_steps, jnp.float32)
    w = (ns / (ns + 1.0)).reshape(1)
    nw = (1.0 / (ns + 1.0)).reshape(1)

    hist_kernel = _make_hist_kernel()
    hists1d = hist_kernel(k_indices)

    updated_avg_norm, updated_clip = _dense_reduce(
        w,
        nw,
        jnp.asarray(avg_norm, jnp.float32).reshape(1),
        jnp.asarray(grad_clip_percent, jnp.float32).reshape(1),
        x,
        updates_flat,
    )

    fd2 = feature_density.reshape(_FD_ROWS, 128)
    hists2 = hists1d.reshape(NW * _FD_ROWS, 128)
    updated_fd = _blend(w, nw, fd2, hists2).reshape(N_FEATURES)

    return (
        jnp.asarray(n_steps + 1),
        updated_avg_norm[0],
        updated_fd,
        updated_clip[0],
    )
